# baseline (speedup 1.0000x reference)
import functools

import jax
import jax.numpy as jnp
import numpy as np
from jax import lax
from jax.experimental import pallas as pl
from jax.experimental.pallas import tpu as pltpu
from jax.experimental.pallas import tpu_sc as plsc

_BATCH = 4096
_HIST = 200
_HP = 208
_NV = _HP // 16
_D = 64
_NC = 2
_NS = 16
_NW = _NC * _NS
_EPW = _BATCH // _NW
_IPW = _EPW * _HP
_OPW = _EPW * _D
_NBUF = 4

_COL_PERM = np.concatenate([
    np.arange(0, 32, 2), np.arange(1, 32, 2),
    np.arange(32, 64, 2), np.arange(33, 64, 2)])


def _unpack2(chunk):
    return plsc.unpack(chunk, format=plsc.PackFormat.INTERLEAVED)


def _sc_pool_body(xp_hbm, table_hbm, out_hbm,
                  idx_v, buf0, buf1, buf2, buf3, row0_v, out_v,
                  sem0, sem1, sem2, sem3):
    bufs = (buf0, buf1, buf2, buf3)
    sems = (sem0, sem1, sem2, sem3)
    wid = lax.axis_index("s") * _NC + lax.axis_index("c")

    pltpu.sync_copy(xp_hbm.at[pl.ds(wid * _IPW, _IPW)], idx_v)
    pltpu.sync_copy(table_hbm.at[pl.ds(0, 8)], row0_v)

    r0a, r0b = _unpack2(row0_v[0, pl.ds(0, 32)])
    r0c, r0d = _unpack2(row0_v[0, pl.ds(32, 32)])
    row0 = (r0a, r0b, r0c, r0d)

    def fire(b, j):
        for k in range(_NV):
            ivec = idx_v[pl.ds(b * _HP + 16 * k, 16)]
            pltpu.async_copy(
                table_hbm.at[ivec], bufs[j].at[pl.ds(16 * k, 16)], sems[j])

    for j in range(_NBUF):
        fire(j, j)

    inv_n = jnp.float32(1.0 / _HIST)

    def elem(i, e):
        b = _NBUF * i + e
        buf = bufs[e]

        cntv = jnp.zeros((16,), jnp.int32)
        for k in range(_NV):
            c = idx_v[pl.ds(b * _HP + 16 * k, 16)]
            cntv += plsc.all_reduce_population_count(c == 0)

        pltpu.make_async_copy(
            table_hbm.at[pl.ds(0, _HP)], buf, sems[e]).wait()

        def row_add(jr, a, unroll=4):
            base = jr * unroll
            for u in range(unroll):
                lo = _unpack2(buf[base + u, pl.ds(0, 32)])
                hi = _unpack2(buf[base + u, pl.ds(32, 32)])
                a = (a[0] + lo[0], a[1] + lo[1],
                     a[2] + hi[0], a[3] + hi[1])
            return a

        acc = (jnp.zeros((16,), jnp.float32),) * 4
        acc = lax.fori_loop(0, _HP // 4, row_add, acc)

        cnt = cntv.astype(jnp.float32)
        for k in range(4):
            val = (acc[k] - cnt * row0[k]) * inv_n
            out_v[pl.ds(b * _D + 16 * k, 16)] = val

        @pl.when(b + _NBUF < _EPW)
        def _(e=e):
            fire(b + _NBUF, e)

    def body(i, carry):
        for e in range(_NBUF):
            elem(i, e)
        return carry

    lax.fori_loop(0, _EPW // _NBUF, body, 0)

    pltpu.sync_copy(out_v, out_hbm.at[pl.ds(wid * _OPW, _OPW)])


_sc_pool = functools.partial(
    pl.kernel,
    out_type=jax.ShapeDtypeStruct((_BATCH * _D,), jnp.float32),
    mesh=plsc.VectorSubcoreMesh(core_axis_name="c", subcore_axis_name="s"),
    compiler_params=pltpu.CompilerParams(
        needs_layout_passes=False, use_tc_tiling_on_sc=False),
    scratch_types=(
        [pltpu.VMEM((_IPW,), jnp.int32)]
        + [pltpu.VMEM((_HP, _D), jnp.bfloat16)] * _NBUF
        + [pltpu.VMEM((8, _D), jnp.bfloat16),
           pltpu.VMEM((_OPW,), jnp.float32)]
        + [pltpu.SemaphoreType.DMA] * _NBUF
    ),
)(_sc_pool_body)


_CB = 16384


def _cast_body(t_ref, o_ref):
    o_ref[...] = t_ref[...].astype(jnp.bfloat16)


def _mlp_body(x_ref, w1_ref, b1_ref, w2_ref, b2_ref, o_ref):
    h = jnp.dot(x_ref[...], w1_ref[...], preferred_element_type=jnp.float32)
    h = jnp.maximum(h + b1_ref[...], 0.0)
    o_ref[...] = (
        jnp.dot(h, w2_ref[...], preferred_element_type=jnp.float32)
        + b2_ref[...])


def kernel(x, table, W1, b1, W2, b2):
    xi = x.astype(jnp.int32)
    xp = jnp.pad(xi, ((0, 0), (0, _HP - _HIST))).reshape(-1)
    vocab = table.shape[0]
    tt = jnp.transpose(table)
    tb_t = pl.pallas_call(
        _cast_body,
        grid=((vocab + _CB - 1) // _CB,),
        in_specs=[pl.BlockSpec((_D, _CB), lambda i: (0, i))],
        out_specs=pl.BlockSpec((_D, _CB), lambda i: (0, i)),
        out_shape=jax.ShapeDtypeStruct((_D, vocab), jnp.bfloat16),
    )(tt)
    tb = jnp.transpose(tb_t)

    pooled = _sc_pool(xp, tb).reshape(_BATCH, _D)

    ncls = W2.shape[1]
    w1p = W1[_COL_PERM, :]
    w2p = jnp.pad(W2, ((0, 0), (0, _D - ncls)))
    b2p = jnp.pad(b2, (0, _D - ncls)).reshape(1, _D)
    out = pl.pallas_call(
        _mlp_body,
        out_shape=jax.ShapeDtypeStruct((_BATCH, _D), jnp.float32),
    )(pooled, w1p, b1.reshape(1, -1), w2p, b2p)
    return out[:, :ncls]


# speedup vs baseline: 1.0659x; 1.0659x over previous
import functools

import jax
import jax.numpy as jnp
import numpy as np
from jax import lax
from jax.experimental import pallas as pl
from jax.experimental.pallas import tpu as pltpu
from jax.experimental.pallas import tpu_sc as plsc

_BATCH = 4096
_HIST = 200
_HP = 208
_NV = _HP // 16
_D = 64
_NC = 2
_NS = 16
_NW = _NC * _NS
_EPW = _BATCH // _NW
_IPW = _EPW * _HP
_OPW = _EPW * _D
_NBUF = 4

_COL_PERM = np.concatenate([
    np.arange(0, 32, 2), np.arange(1, 32, 2),
    np.arange(32, 64, 2), np.arange(33, 64, 2)])


def _unpack2(chunk):
    return plsc.unpack(chunk, format=plsc.PackFormat.INTERLEAVED)


def _sc_pool_body(xp_hbm, table_hbm, out_hbm,
                  idx_v, buf0, buf1, buf2, buf3, row0_v, out_v,
                  sem0, sem1, sem2, sem3):
    bufs = (buf0, buf1, buf2, buf3)
    sems = (sem0, sem1, sem2, sem3)
    wid = lax.axis_index("s") * _NC + lax.axis_index("c")

    pltpu.sync_copy(xp_hbm.at[pl.ds(wid * _IPW, _IPW)], idx_v)
    pltpu.sync_copy(table_hbm.at[pl.ds(0, 8)], row0_v)

    r0a, r0b = _unpack2(row0_v[0, pl.ds(0, 32)])
    r0c, r0d = _unpack2(row0_v[0, pl.ds(32, 32)])
    row0 = (r0a, r0b, r0c, r0d)

    def fire(b, j):
        for k in range(_NV):
            ivec = idx_v[pl.ds(b * _HP + 16 * k, 16)]
            pltpu.async_copy(
                table_hbm.at[ivec], bufs[j].at[pl.ds(16 * k, 16)], sems[j])

    for j in range(_NBUF):
        fire(j, j)

    inv_n = jnp.float32(1.0 / _HIST)

    def elem(i, e):
        b = _NBUF * i + e
        buf = bufs[e]

        cntv = jnp.zeros((16,), jnp.int32)
        for k in range(_NV):
            c = idx_v[pl.ds(b * _HP + 16 * k, 16)]
            cntv += plsc.all_reduce_population_count(c == 0)

        pltpu.make_async_copy(
            table_hbm.at[pl.ds(0, _HP)], buf, sems[e]).wait()

        def row_add(jr, a, unroll=4):
            base = jr * unroll
            for u in range(unroll):
                lo = _unpack2(buf[base + u, pl.ds(0, 32)])
                hi = _unpack2(buf[base + u, pl.ds(32, 32)])
                a = (a[0] + lo[0], a[1] + lo[1],
                     a[2] + hi[0], a[3] + hi[1])
            return a

        acc = (jnp.zeros((16,), jnp.float32),) * 4
        acc = lax.fori_loop(0, _HP // 4, row_add, acc)

        cnt = cntv.astype(jnp.float32)
        for k in range(4):
            val = (acc[k] - cnt * row0[k]) * inv_n
            out_v[pl.ds(b * _D + 16 * k, 16)] = val

        @pl.when(b + _NBUF < _EPW)
        def _(e=e):
            fire(b + _NBUF, e)

    def body(i, carry):
        for e in range(_NBUF):
            elem(i, e)
        return carry

    lax.fori_loop(0, _EPW // _NBUF, body, 0)

    pltpu.sync_copy(out_v, out_hbm.at[pl.ds(wid * _OPW, _OPW)])


_sc_pool = functools.partial(
    pl.kernel,
    out_type=jax.ShapeDtypeStruct((_BATCH * _D,), jnp.float32),
    mesh=plsc.VectorSubcoreMesh(core_axis_name="c", subcore_axis_name="s"),
    compiler_params=pltpu.CompilerParams(
        needs_layout_passes=False, use_tc_tiling_on_sc=False),
    scratch_types=(
        [pltpu.VMEM((_IPW,), jnp.int32)]
        + [pltpu.VMEM((_HP, _D), jnp.bfloat16)] * _NBUF
        + [pltpu.VMEM((8, _D), jnp.bfloat16),
           pltpu.VMEM((_OPW,), jnp.float32)]
        + [pltpu.SemaphoreType.DMA] * _NBUF
    ),
)(_sc_pool_body)


def _mlp_body(x_ref, w1_ref, b1_ref, w2_ref, b2_ref, o_ref):
    h = jnp.dot(x_ref[...], w1_ref[...], preferred_element_type=jnp.float32)
    h = jnp.maximum(h + b1_ref[...], 0.0)
    o_ref[...] = (
        jnp.dot(h, w2_ref[...], preferred_element_type=jnp.float32)
        + b2_ref[...])


def kernel(x, table, W1, b1, W2, b2):
    xi = x.astype(jnp.int32)
    xp = jnp.pad(xi, ((0, 0), (0, _HP - _HIST))).reshape(-1)
    tb = jnp.transpose(jnp.transpose(table).astype(jnp.bfloat16))

    pooled = _sc_pool(xp, tb).reshape(_BATCH, _D)

    ncls = W2.shape[1]
    w1p = W1[_COL_PERM, :]
    w2p = jnp.pad(W2, ((0, 0), (0, _D - ncls)))
    b2p = jnp.pad(b2, (0, _D - ncls)).reshape(1, _D)
    out = pl.pallas_call(
        _mlp_body,
        out_shape=jax.ShapeDtypeStruct((_BATCH, _D), jnp.float32),
    )(pooled, w1p, b1.reshape(1, -1), w2p, b2p)
    return out[:, :ncls]


# speedup vs baseline: 1.0693x; 1.0032x over previous
import functools

import jax
import jax.numpy as jnp
import numpy as np
from jax import lax
from jax.experimental import pallas as pl
from jax.experimental.pallas import tpu as pltpu
from jax.experimental.pallas import tpu_sc as plsc

_BATCH = 4096
_HIST = 200
_HP = 208
_NV = _HP // 16
_D = 64
_NC = 2
_NS = 16
_NW = _NC * _NS
_EPW = _BATCH // _NW
_IPW = _EPW * _HP
_OPW = _EPW * _D
_NBUF = 4

_COL_PERM = np.concatenate([
    np.arange(0, 32, 2), np.arange(1, 32, 2),
    np.arange(32, 64, 2), np.arange(33, 64, 2)])


def _unpack2(chunk):
    return plsc.unpack(chunk, format=plsc.PackFormat.INTERLEAVED)


def _sc_pool_body(xp_hbm, table_hbm, out_hbm,
                  idx_v, buf0, buf1, buf2, buf3, row0_v, out_v,
                  sem0, sem1, sem2, sem3):
    bufs = (buf0, buf1, buf2, buf3)
    sems = (sem0, sem1, sem2, sem3)
    wid = lax.axis_index("s") * _NC + lax.axis_index("c")

    pltpu.sync_copy(xp_hbm.at[pl.ds(wid * _IPW, _IPW)], idx_v)
    pltpu.sync_copy(table_hbm.at[pl.ds(0, 8)], row0_v)

    r0a, r0b = _unpack2(row0_v[0, pl.ds(0, 32)])
    r0c, r0d = _unpack2(row0_v[0, pl.ds(32, 32)])
    row0 = (r0a, r0b, r0c, r0d)

    def fire(b, j):
        for k in range(_NV):
            ivec = idx_v[pl.ds(b * _HP + 16 * k, 16)]
            pltpu.async_copy(
                table_hbm.at[ivec], bufs[j].at[pl.ds(16 * k, 16)], sems[j])

    for j in range(_NBUF):
        fire(j, j)

    inv_n = jnp.float32(1.0 / _HIST)

    def elem(i, e):
        b = _NBUF * i + e
        buf = bufs[e]

        cntv = jnp.zeros((16,), jnp.int32)
        for k in range(_NV):
            c = idx_v[pl.ds(b * _HP + 16 * k, 16)]
            cntv += plsc.all_reduce_population_count(c == 0)

        pltpu.make_async_copy(
            table_hbm.at[pl.ds(0, _HP)], buf, sems[e]).wait()

        def row_add(jr, a, unroll=4):
            base = jr * unroll
            for u in range(unroll):
                lo = _unpack2(buf[base + u, pl.ds(0, 32)])
                hi = _unpack2(buf[base + u, pl.ds(32, 32)])
                a = (a[0] + lo[0], a[1] + lo[1],
                     a[2] + hi[0], a[3] + hi[1])
            return a

        acc = (jnp.zeros((16,), jnp.float32),) * 4
        acc = lax.fori_loop(0, _HP // 4, row_add, acc)

        cnt = cntv.astype(jnp.float32)
        for k in range(4):
            val = (acc[k] - cnt * row0[k]) * inv_n
            out_v[pl.ds(b * _D + 16 * k, 16)] = val

        @pl.when(b + _NBUF < _EPW)
        def _(e=e):
            fire(b + _NBUF, e)

    def body(i, carry):
        for e in range(_NBUF):
            elem(i, e)
        return carry

    lax.fori_loop(0, _EPW // _NBUF, body, 0)

    pltpu.sync_copy(out_v, out_hbm.at[pl.ds(wid * _OPW, _OPW)])


_sc_pool = functools.partial(
    pl.kernel,
    out_type=jax.ShapeDtypeStruct((_BATCH * _D,), jnp.float32),
    mesh=plsc.VectorSubcoreMesh(core_axis_name="c", subcore_axis_name="s"),
    compiler_params=pltpu.CompilerParams(
        needs_layout_passes=False, use_tc_tiling_on_sc=False),
    scratch_types=(
        [pltpu.VMEM((_IPW,), jnp.int32)]
        + [pltpu.VMEM((_HP, _D), jnp.bfloat16)] * _NBUF
        + [pltpu.VMEM((8, _D), jnp.bfloat16),
           pltpu.VMEM((_OPW,), jnp.float32)]
        + [pltpu.SemaphoreType.DMA] * _NBUF
    ),
)(_sc_pool_body)


def _mlp_body(x_ref, w1_ref, b1_ref, w2_ref, b2_ref, o_ref):
    h = jnp.dot(x_ref[...], w1_ref[...], preferred_element_type=jnp.float32)
    h = jnp.maximum(h + b1_ref[...], 0.0)
    o_ref[...] = (
        jnp.dot(h, w2_ref[...], preferred_element_type=jnp.float32)
        + b2_ref[...])


def kernel(x, table, W1, b1, W2, b2):
    xi = x.astype(jnp.int32)
    xp = jnp.pad(xi, ((0, 0), (0, _HP - _HIST))).reshape(-1)
    tb = table.astype(jnp.bfloat16)

    pooled = _sc_pool(xp, tb).reshape(_BATCH, _D)

    ncls = W2.shape[1]
    w1p = W1[_COL_PERM, :]
    w2p = jnp.pad(W2, ((0, 0), (0, _D - ncls)))
    b2p = jnp.pad(b2, (0, _D - ncls)).reshape(1, _D)
    out = pl.pallas_call(
        _mlp_body,
        out_shape=jax.ShapeDtypeStruct((_BATCH, _D), jnp.float32),
    )(pooled, w1p, b1.reshape(1, -1), w2p, b2p)
    return out[:, :ncls]
